# baseline (speedup 1.0000x reference)
import functools

import jax
import jax.numpy as jnp
from jax import lax
from jax.experimental import pallas as pl
from jax.experimental.pallas import tpu as pltpu
from jax.experimental.pallas import tpu_sc as plsc

BATCH = 4096
SEQ = 200
EMBED = 64
VOCAB = 1000000
NUM_CORES = 2
NUM_SUBCORES = 16
NUM_WORKERS = NUM_CORES * NUM_SUBCORES
SEQS_PER_WORKER = BATCH // NUM_WORKERS
BLOCKS_PER_WORKER = SEQS_PER_WORKER // 8
LANES = 16
PAIR = 2 * EMBED
G0 = 128
G1 = SEQ - G0


def _body(idx_hbm, tok2_hbm, pos2_hbm, out_hbm, idx2_v, pair2_v, rows2_v,
          out2_v, pos2_v, sem_i, sg0, sg1, so0, so1):
    wid = lax.axis_index("s") * NUM_CORES + lax.axis_index("c")
    blk0 = wid * BLOCKS_PER_WORKER
    sg = [sg0, sg1]
    so = [so0, so1]

    pltpu.sync_copy(pos2_hbm, pos2_v)

    def shift_seq(p, bp, jb):
        def shift_body(q, _):
            sl = pl.ds(q * LANES, LANES)
            pair2_v[p, sl] = lax.shift_right_logical(idx2_v[bp, jb, sl], 1)
            return ()

        lax.fori_loop(0, 12, shift_body, ())
        tl = pl.ds(SEQ - LANES, LANES)
        pair2_v[p, tl] = lax.shift_right_logical(idx2_v[bp, jb, tl], 1)

    def gather_copies(p):
        return (
            pltpu.make_async_copy(
                tok2_hbm.at[pair2_v.at[p, pl.ds(0, G0)]],
                rows2_v.at[p, pl.ds(0, G0)],
                sg[p],
            ),
            pltpu.make_async_copy(
                tok2_hbm.at[pair2_v.at[p, pl.ds(G0, G1)]],
                rows2_v.at[p, pl.ds(G0, G1)],
                sg[p],
            ),
        )

    def gather_start(p):
        for cp in gather_copies(p):
            cp.start()

    def gather_wait(p):
        for cp in gather_copies(p):
            cp.wait()

    def idx_copy(blk, bp):
        return pltpu.make_async_copy(
            idx_hbm.at[pl.ds(blk * 8, 8)], idx2_v.at[bp], sem_i
        )

    def out_copy(b, p):
        return pltpu.make_async_copy(out2_v.at[p], out_hbm.at[b], so[p])

    def add_16rows(p, bp, jb, b0, js):
        hv = (idx2_v[bp, jb, pl.ds(b0, LANES)] & 1) * EMBED
        for j in js:
            r = b0 + j
            r3 = b0 // 8 + j // 8
            h = pl.multiple_of(hv[j], EMBED)
            r2 = b0 // 2 + j // 2
            for cc in range(EMBED // LANES):
                out2_v[p, r3, j % 8, pl.ds(cc * LANES, LANES)] = (
                    rows2_v[p, r, pl.ds(h + cc * LANES, LANES)]
                    + pos2_v[r2, pl.ds((j % 2) * EMBED + cc * LANES, LANES)]
                )

    def add_seq(p, bp, jb):
        def add_body(q, _):
            add_16rows(p, bp, jb, q * LANES, range(16))
            return ()

        lax.fori_loop(0, 12, add_body, ())
        add_16rows(p, bp, jb, SEQ - LANES, range(8, 16))

    pltpu.sync_copy(idx_hbm.at[pl.ds(blk0 * 8, 8)], idx2_v.at[0])
    shift_seq(0, 0, 0)
    gather_start(0)

    def block_body(c, _):
        bp = c & 1

        @pl.when(c < BLOCKS_PER_WORKER - 1)
        def _():
            idx_copy(blk0 + c + 1, 1 - bp).start()

        def quad_body(u, _):
            for v in range(2):
                jb = 2 * u + v
                s = 8 * c + jb
                if v == 0:
                    shift_seq(1, bp, jb + 1)
                    gather_start(1)
                else:
                    @pl.when(u < 3)
                    def _():
                        shift_seq(0, bp, jb + 1)
                        gather_start(0)

                    @pl.when((u == 3) & (c < BLOCKS_PER_WORKER - 1))
                    def _():
                        idx_copy(blk0 + c + 1, 1 - bp).wait()
                        shift_seq(0, 1 - bp, 0)
                        gather_start(0)

                gather_wait(v)

                @pl.when(s >= 2)
                def _():
                    out_copy(blk0 * 8 + s, v).wait()

                add_seq(v, bp, jb)
                out_copy(blk0 * 8 + s, v).start()
            return ()

        lax.fori_loop(0, 4, quad_body, ())
        return ()

    lax.fori_loop(0, BLOCKS_PER_WORKER, block_body, ())

    out_copy(blk0 * 8, 0).wait()
    out_copy(blk0 * 8, 1).wait()


@jax.jit
def kernel(inputs, token_table, position_table):
    tok2 = jnp.concatenate(
        [token_table[0::2], token_table[1::2]], axis=1
    )
    pos2 = position_table.reshape(SEQ // 2, PAIR)
    mesh = plsc.VectorSubcoreMesh(
        core_axis_name="c", subcore_axis_name="s", num_cores=NUM_CORES,
        num_subcores=NUM_SUBCORES,
    )
    out4 = pl.kernel(
        _body,
        out_type=jax.ShapeDtypeStruct((BATCH, SEQ // 8, 8, EMBED),
                                      jnp.float32),
        mesh=mesh,
        scratch_types=[
            pltpu.VMEM((2, 8, SEQ), jnp.int32),
            pltpu.VMEM((2, SEQ), jnp.int32),
            pltpu.VMEM((2, SEQ, PAIR), jnp.float32),
            pltpu.VMEM((2, SEQ // 8, 8, EMBED), jnp.float32),
            pltpu.VMEM((SEQ // 2, PAIR), jnp.float32),
            pltpu.SemaphoreType.DMA,
            pltpu.SemaphoreType.DMA,
            pltpu.SemaphoreType.DMA,
            pltpu.SemaphoreType.DMA,
            pltpu.SemaphoreType.DMA,
        ],
    )(inputs, tok2, pos2)
    return out4.reshape(BATCH, SEQ, EMBED)


# speedup vs baseline: 7.0925x; 7.0925x over previous
import functools

import jax
import jax.numpy as jnp
from jax import lax
from jax.experimental import pallas as pl
from jax.experimental.pallas import tpu as pltpu
from jax.experimental.pallas import tpu_sc as plsc

BATCH = 4096
SEQ = 200
EMBED = 64
VOCAB = 1000000
NUM_CORES = 2
NUM_SUBCORES = 16
NUM_WORKERS = NUM_CORES * NUM_SUBCORES
SEQS_PER_WORKER = BATCH // NUM_WORKERS
BLOCKS_PER_WORKER = SEQS_PER_WORKER // 8
LANES = 16
PAIR = 2 * EMBED
G0 = 128
G1 = SEQ - G0


def _body(idx_hbm, tok2_hbm, pos2_hbm, out_hbm, idx2_v, pair2_v, rows2_v,
          out2_v, pos2_v, sem_i, sg0, sg1, so0, so1):
    wid = lax.axis_index("s") * NUM_CORES + lax.axis_index("c")
    blk0 = wid * BLOCKS_PER_WORKER
    sg = [sg0, sg1]
    so = [so0, so1]

    pltpu.sync_copy(pos2_hbm, pos2_v)

    def shift_seq(p, bp, jb):
        def shift_body(q, _):
            sl = pl.ds(q * LANES, LANES)
            pair2_v[p, sl] = lax.shift_right_logical(idx2_v[bp, jb, sl], 1)
            return ()

        lax.fori_loop(0, 12, shift_body, ())
        tl = pl.ds(SEQ - LANES, LANES)
        pair2_v[p, tl] = lax.shift_right_logical(idx2_v[bp, jb, tl], 1)

    def gather_copies(p):
        return (
            pltpu.make_async_copy(
                tok2_hbm.at[pair2_v.at[p, pl.ds(0, G0)]],
                rows2_v.at[p, pl.ds(0, G0)],
                sg[p],
            ),
            pltpu.make_async_copy(
                tok2_hbm.at[pair2_v.at[p, pl.ds(G0, G1)]],
                rows2_v.at[p, pl.ds(G0, G1)],
                sg[p],
            ),
        )

    def gather_start(p):
        for cp in gather_copies(p):
            cp.start()

    def gather_wait(p):
        for cp in gather_copies(p):
            cp.wait()

    def idx_copy(blk, bp):
        return pltpu.make_async_copy(
            idx_hbm.at[pl.ds(blk * 8, 8)], idx2_v.at[bp], sem_i
        )

    def out_copy(b, p):
        return pltpu.make_async_copy(out2_v.at[p], out_hbm.at[b], so[p])

    def add_16rows(p, bp, jb, b0, js):
        hv = (idx2_v[bp, jb, pl.ds(b0, LANES)] & 1) * EMBED
        for j in js:
            r = b0 + j
            r3 = b0 // 8 + j // 8
            h = pl.multiple_of(hv[j], EMBED)
            r2 = b0 // 2 + j // 2
            for cc in range(EMBED // LANES):
                out2_v[p, r3, j % 8, pl.ds(cc * LANES, LANES)] = (
                    rows2_v[p, r, pl.ds(h + cc * LANES, LANES)]
                    + pos2_v[r2, pl.ds((j % 2) * EMBED + cc * LANES, LANES)]
                )

    def add_rows(p, bp, jb, q_lo, q_hi):
        def add_body(q, _):
            add_16rows(p, bp, jb, q * LANES, range(16))
            return ()

        lax.fori_loop(q_lo, q_hi, add_body, ())

    def add_tail(p, bp, jb):
        add_16rows(p, bp, jb, SEQ - LANES, range(8, 16))

    pltpu.sync_copy(idx_hbm.at[pl.ds(blk0 * 8, 8)], idx2_v.at[0])
    shift_seq(0, 0, 0)
    gather_start(0)

    def block_body(c, _):
        bp = c & 1

        @pl.when(c < BLOCKS_PER_WORKER - 1)
        def _():
            idx_copy(blk0 + c + 1, 1 - bp).start()

        def quad_body(u, _):
            for v in range(2):
                jb = 2 * u + v
                s = 8 * c + jb
                if v == 0:
                    shift_seq(1, bp, jb + 1)
                    gather_start(1)
                else:
                    @pl.when(u < 3)
                    def _():
                        shift_seq(0, bp, jb + 1)
                        gather_start(0)

                    @pl.when((u == 3) & (c < BLOCKS_PER_WORKER - 1))
                    def _():
                        idx_copy(blk0 + c + 1, 1 - bp).wait()
                        shift_seq(0, 1 - bp, 0)
                        gather_start(0)

                @pl.when(s >= 2)
                def _():
                    out_copy(blk0 * 8 + s, v).wait()

                cp0, cp1 = gather_copies(v)
                cp0.wait()
                add_rows(v, bp, jb, 0, G0 // LANES)
                cp1.wait()
                add_rows(v, bp, jb, G0 // LANES, 12)
                add_tail(v, bp, jb)
                out_copy(blk0 * 8 + s, v).start()
            return ()

        lax.fori_loop(0, 4, quad_body, ())
        return ()

    lax.fori_loop(0, BLOCKS_PER_WORKER, block_body, ())

    out_copy(blk0 * 8, 0).wait()
    out_copy(blk0 * 8, 1).wait()


@jax.jit
def kernel(inputs, token_table, position_table):
    tok2 = token_table.reshape(VOCAB // 2, PAIR)
    pos2 = position_table.reshape(SEQ // 2, PAIR)
    mesh = plsc.VectorSubcoreMesh(
        core_axis_name="c", subcore_axis_name="s", num_cores=NUM_CORES,
        num_subcores=NUM_SUBCORES,
    )
    out4 = pl.kernel(
        _body,
        out_type=jax.ShapeDtypeStruct((BATCH, SEQ // 8, 8, EMBED),
                                      jnp.float32),
        mesh=mesh,
        scratch_types=[
            pltpu.VMEM((2, 8, SEQ), jnp.int32),
            pltpu.VMEM((2, SEQ), jnp.int32),
            pltpu.VMEM((2, SEQ, PAIR), jnp.float32),
            pltpu.VMEM((2, SEQ // 8, 8, EMBED), jnp.float32),
            pltpu.VMEM((SEQ // 2, PAIR), jnp.float32),
            pltpu.SemaphoreType.DMA,
            pltpu.SemaphoreType.DMA,
            pltpu.SemaphoreType.DMA,
            pltpu.SemaphoreType.DMA,
            pltpu.SemaphoreType.DMA,
        ],
    )(inputs, tok2, pos2)
    return out4.reshape(BATCH, SEQ, EMBED)
